# baseline (speedup 1.0000x reference)
import jax
import jax.numpy as jnp
from jax.experimental import pallas as pl
from jax.experimental.pallas import tpu as pltpu

B_TOTAL = 262144
K = 128
BLK = 2048
NSTEP = B_TOTAL // BLK


def _outer(a, b):
    return jax.lax.dot_general(
        a, b, (((0,), (0,)), ((), ())), preferred_element_type=jnp.float32)


def _fused_kernel(f_ref, w_ref, o_ref, acc_ref, vec_ref):
    j = pl.program_id(0)

    @pl.when(j == 0)
    def _():
        acc_ref[...] = jnp.zeros_like(acc_ref)
        vec_ref[...] = jnp.zeros_like(vec_ref)

    HB = BLK // 2
    for half in range(2):
        f = f_ref[half * HB:(half + 1) * HB, :]
        w_row = w_ref[0, :, half * HB:(half + 1) * HB]

        h32 = pltpu.bitcast(
            pltpu.bitcast(f, jnp.int32) & jnp.int32(-65536), jnp.float32)
        l32 = f - h32
        rhs = jnp.concatenate([h32, l32], axis=1).astype(jnp.bfloat16)
        h = rhs[:, :K]

        ht = h.T
        wb = w_row.astype(jnp.bfloat16)
        hwt = ht * wb
        ones_row = jnp.ones((1, HB), jnp.bfloat16)
        zpad = jnp.zeros((6, HB), jnp.bfloat16)
        aux = jnp.concatenate([ones_row, wb, zpad], axis=0)
        lhs = jnp.concatenate([ht, hwt, aux], axis=0)

        m = jax.lax.dot_general(
            lhs, rhs, (((1,), (0,)), ((), ())),
            preferred_element_type=jnp.float32)
        acc_ref[...] += m

        n1 = jnp.sum(w_row, keepdims=True)
        vec_ref[0:1, :] += jnp.broadcast_to(n1, (1, K))

    @pl.when(j == NSTEP - 1)
    def _():
        MH = acc_ref[:K, :]
        MW = acc_ref[K:2 * K, :]
        srow = acc_ref[2 * K:2 * K + 1, :]
        wrow = acc_ref[2 * K + 1:2 * K + 2, :]
        C = MH[:, K:]
        D = MW[:, K:]
        GA = MH[:, :K] + C + C.T
        G1w = MW[:, :K] + D + D.T
        G0 = GA - G1w
        suma_ = srow[:, :K] + srow[:, K:]
        sum1w = wrow[:, :K] + wrow[:, K:]
        sum0_ = suma_ - sum1w
        n1_ = vec_ref[0:1, 0:1]
        n0_ = float(B_TOTAL) - n1_
        r0 = 1.0 / n0_
        r1 = 1.0 / n1_
        mu1 = sum0_ * r0
        mu2 = sum1w * r1

        S1 = G0 - _outer(sum0_, sum0_) * r0
        S2 = (G1w - _outer(mu1, sum1w) - _outer(sum1w, mu1)
              + n1_ * _outer(mu1, mu1))

        ri = jax.lax.broadcasted_iota(jnp.int32, (K, 1), 0)
        ci = jax.lax.broadcasted_iota(jnp.int32, (1, K), 1)
        eye = (ri == ci).astype(jnp.float32)
        Sf1 = eye + S1 * r0
        Sf2 = eye + S2 * r1

        def gj_step(i, carry):
            M2, Inv, ld2c, M1, ld1c = carry
            ej_row = (ci == i).astype(jnp.float32)
            ej_col = (ri == i).astype(jnp.float32)
            rowm2 = jnp.sum(M2 * ej_col, axis=0, keepdims=True)
            rowi = jnp.sum(Inv * ej_col, axis=0, keepdims=True)
            rowm1 = jnp.sum(M1 * ej_col, axis=0, keepdims=True)
            colm2 = jnp.sum(M2 * ej_row, axis=1, keepdims=True)
            colm1 = jnp.sum(M1 * ej_row, axis=1, keepdims=True)
            p2 = jnp.sum(rowm2 * ej_row, axis=1, keepdims=True)
            p1 = jnp.sum(rowm1 * ej_row, axis=1, keepdims=True)
            rp2 = 1.0 / p2
            rp1 = 1.0 / p1
            cm2 = colm2 - ej_col
            cm1 = colm1 - ej_col
            M2 = M2 - cm2 * (rowm2 * rp2)
            Inv = Inv - cm2 * (rowi * rp2)
            M1 = M1 - cm1 * (rowm1 * rp1)
            ld2c = ld2c + (jnp.log2(p2) - 1.0)
            ld1c = ld1c + (jnp.log2(p1) - 1.0)
            return (M2, Inv, ld2c, M1, ld1c)

        ld0 = jnp.zeros((1, 1), jnp.float32)
        _, inv2, ld2, _, ld1 = jax.lax.fori_loop(
            0, K, gj_step, (Sf2, eye, ld0, Sf1, ld0))

        d = mu1 - mu2
        quad = jnp.sum(inv2 * _outer(d, d), keepdims=True)[0:1, 0:1]
        trd = jnp.sum(inv2 * (Sf1 - Sf2), keepdims=True)[0:1, 0:1]
        o_ref[...] = 0.5 * ((ld2 - ld1) + quad + trd)


def kernel(feature, label):
    labf = label.astype(jnp.float32).reshape(NSTEP, 1, BLK)
    out = pl.pallas_call(
        _fused_kernel,
        grid=(NSTEP,),
        in_specs=[
            pl.BlockSpec((BLK, K), lambda j: (j, 0)),
            pl.BlockSpec((1, 1, BLK), lambda j: (j, 0, 0)),
        ],
        out_specs=pl.BlockSpec((1, 1), lambda j: (0, 0)),
        out_shape=jax.ShapeDtypeStruct((1, 1), jnp.float32),
        scratch_shapes=[
            pltpu.VMEM((2 * K + 8, 2 * K), jnp.float32),
            pltpu.VMEM((8, K), jnp.float32),
        ],
        compiler_params=pltpu.CompilerParams(
            dimension_semantics=("arbitrary",),
        ),
    )(feature, labf)
    return out


# speedup vs baseline: 1.3624x; 1.3624x over previous
import jax
import jax.numpy as jnp
from jax.experimental import pallas as pl
from jax.experimental.pallas import tpu as pltpu

B_TOTAL = 262144
K = 128
BLK = 4096
NSTEP = B_TOTAL // BLK


def _outer(a, b):
    return jax.lax.dot_general(
        a, b, (((0,), (0,)), ((), ())), preferred_element_type=jnp.float32)


def _fused_kernel(f_ref, w_ref, o_ref, acc_ref, vec_ref):
    j = pl.program_id(0)

    @pl.when(j == 0)
    def _():
        acc_ref[...] = jnp.zeros_like(acc_ref)
        vec_ref[...] = jnp.zeros_like(vec_ref)

    HB = BLK // 4
    for half in range(4):
        f = f_ref[half * HB:(half + 1) * HB, :]
        w_row = w_ref[0, :, half * HB:(half + 1) * HB]

        h32 = pltpu.bitcast(
            pltpu.bitcast(f, jnp.int32) & jnp.int32(-65536), jnp.float32)
        l32 = f - h32
        rhs = jnp.concatenate([h32, l32], axis=1).astype(jnp.bfloat16)
        h = rhs[:, :K]

        ht = h.T
        wb = w_row.astype(jnp.bfloat16)
        hwt = ht * wb
        ones_row = jnp.ones((1, HB), jnp.bfloat16)
        zpad = jnp.zeros((6, HB), jnp.bfloat16)
        aux = jnp.concatenate([ones_row, wb, zpad], axis=0)
        lhs = jnp.concatenate([ht, hwt, aux], axis=0)

        m = jax.lax.dot_general(
            lhs, rhs, (((1,), (0,)), ((), ())),
            preferred_element_type=jnp.float32)
        acc_ref[...] += m

        n1 = jnp.sum(w_row, keepdims=True)
        vec_ref[0:1, :] += jnp.broadcast_to(n1, (1, K))

    @pl.when(j == NSTEP - 1)
    def _():
        MH = acc_ref[:K, :]
        MW = acc_ref[K:2 * K, :]
        srow = acc_ref[2 * K:2 * K + 1, :]
        wrow = acc_ref[2 * K + 1:2 * K + 2, :]
        C = MH[:, K:]
        D = MW[:, K:]
        GA = MH[:, :K] + C + C.T
        G1w = MW[:, :K] + D + D.T
        G0 = GA - G1w
        suma_ = srow[:, :K] + srow[:, K:]
        sum1w = wrow[:, :K] + wrow[:, K:]
        sum0_ = suma_ - sum1w
        n1_ = vec_ref[0:1, 0:1]
        n0_ = float(B_TOTAL) - n1_
        r0 = 1.0 / n0_
        r1 = 1.0 / n1_
        mu1 = sum0_ * r0
        mu2 = sum1w * r1

        S1 = G0 - _outer(sum0_, sum0_) * r0
        S2 = (G1w - _outer(mu1, sum1w) - _outer(sum1w, mu1)
              + n1_ * _outer(mu1, mu1))

        ri = jax.lax.broadcasted_iota(jnp.int32, (K, 1), 0)
        ci = jax.lax.broadcasted_iota(jnp.int32, (1, K), 1)
        eye = (ri == ci).astype(jnp.float32)
        Sf1 = eye + S1 * r0
        Sf2 = eye + S2 * r1

        def gj_step(i, carry):
            M2, Inv, ld2c, M1, ld1c = carry
            ej_row = (ci == i).astype(jnp.float32)
            ej_col = (ri == i).astype(jnp.float32)
            rowm2 = jnp.sum(M2 * ej_col, axis=0, keepdims=True)
            rowi = jnp.sum(Inv * ej_col, axis=0, keepdims=True)
            rowm1 = jnp.sum(M1 * ej_col, axis=0, keepdims=True)
            colm2 = jnp.sum(M2 * ej_row, axis=1, keepdims=True)
            colm1 = jnp.sum(M1 * ej_row, axis=1, keepdims=True)
            p2 = jnp.sum(rowm2 * ej_row, axis=1, keepdims=True)
            p1 = jnp.sum(rowm1 * ej_row, axis=1, keepdims=True)
            rp2 = 1.0 / p2
            rp1 = 1.0 / p1
            cm2 = colm2 - ej_col
            cm1 = colm1 - ej_col
            M2 = M2 - cm2 * (rowm2 * rp2)
            Inv = Inv - cm2 * (rowi * rp2)
            M1 = M1 - cm1 * (rowm1 * rp1)
            ld2c = ld2c + (jnp.log2(p2) - 1.0)
            ld1c = ld1c + (jnp.log2(p1) - 1.0)
            return (M2, Inv, ld2c, M1, ld1c)

        ld0 = jnp.zeros((1, 1), jnp.float32)
        _, inv2, ld2, _, ld1 = jax.lax.fori_loop(
            0, K, gj_step, (Sf2, eye, ld0, Sf1, ld0))

        d = mu1 - mu2
        quad = jnp.sum(inv2 * _outer(d, d), keepdims=True)[0:1, 0:1]
        trd = jnp.sum(inv2 * (Sf1 - Sf2), keepdims=True)[0:1, 0:1]
        o_ref[...] = 0.5 * ((ld2 - ld1) + quad + trd)


def kernel(feature, label):
    labf = label.astype(jnp.float32).reshape(NSTEP, 1, BLK)
    out = pl.pallas_call(
        _fused_kernel,
        grid=(NSTEP,),
        in_specs=[
            pl.BlockSpec((BLK, K), lambda j: (j, 0)),
            pl.BlockSpec((1, 1, BLK), lambda j: (j, 0, 0)),
        ],
        out_specs=pl.BlockSpec((1, 1), lambda j: (0, 0)),
        out_shape=jax.ShapeDtypeStruct((1, 1), jnp.float32),
        scratch_shapes=[
            pltpu.VMEM((2 * K + 8, 2 * K), jnp.float32),
            pltpu.VMEM((8, K), jnp.float32),
        ],
        compiler_params=pltpu.CompilerParams(
            dimension_semantics=("arbitrary",),
        ),
    )(feature, labf)
    return out


# speedup vs baseline: 1.6768x; 1.2308x over previous
import jax
import jax.numpy as jnp
from jax.experimental import pallas as pl
from jax.experimental.pallas import tpu as pltpu

B_TOTAL = 262144
K = 128
BLK = 8192
NSTEP = B_TOTAL // BLK


def _outer(a, b):
    return jax.lax.dot_general(
        a, b, (((0,), (0,)), ((), ())), preferred_element_type=jnp.float32)


def _fused_kernel(f_ref, w_ref, o_ref, acc_ref, vec_ref):
    j = pl.program_id(0)

    @pl.when(j == 0)
    def _():
        acc_ref[...] = jnp.zeros_like(acc_ref)
        vec_ref[...] = jnp.zeros_like(vec_ref)

    HB = BLK // 8
    for half in range(8):
        f = f_ref[half * HB:(half + 1) * HB, :]
        w_row = w_ref[0, :, half * HB:(half + 1) * HB]

        h32 = pltpu.bitcast(
            pltpu.bitcast(f, jnp.int32) & jnp.int32(-65536), jnp.float32)
        l32 = f - h32
        rhs = jnp.concatenate([h32, l32], axis=1).astype(jnp.bfloat16)
        h = rhs[:, :K]

        ht = h.T
        wb = w_row.astype(jnp.bfloat16)
        hwt = ht * wb
        ones_row = jnp.ones((1, HB), jnp.bfloat16)
        zpad = jnp.zeros((6, HB), jnp.bfloat16)
        aux = jnp.concatenate([ones_row, wb, zpad], axis=0)
        lhs = jnp.concatenate([ht, hwt, aux], axis=0)

        m = jax.lax.dot_general(
            lhs, rhs, (((1,), (0,)), ((), ())),
            preferred_element_type=jnp.float32)
        acc_ref[...] += m

        n1 = jnp.sum(w_row, keepdims=True)
        vec_ref[0:1, :] += jnp.broadcast_to(n1, (1, K))

    @pl.when(j == NSTEP - 1)
    def _():
        MH = acc_ref[:K, :]
        MW = acc_ref[K:2 * K, :]
        srow = acc_ref[2 * K:2 * K + 1, :]
        wrow = acc_ref[2 * K + 1:2 * K + 2, :]
        C = MH[:, K:]
        D = MW[:, K:]
        GA = MH[:, :K] + C + C.T
        G1w = MW[:, :K] + D + D.T
        G0 = GA - G1w
        suma_ = srow[:, :K] + srow[:, K:]
        sum1w = wrow[:, :K] + wrow[:, K:]
        sum0_ = suma_ - sum1w
        n1_ = vec_ref[0:1, 0:1]
        n0_ = float(B_TOTAL) - n1_
        r0 = 1.0 / n0_
        r1 = 1.0 / n1_
        mu1 = sum0_ * r0
        mu2 = sum1w * r1

        S1 = G0 - _outer(sum0_, sum0_) * r0
        S2 = (G1w - _outer(mu1, sum1w) - _outer(sum1w, mu1)
              + n1_ * _outer(mu1, mu1))

        ri = jax.lax.broadcasted_iota(jnp.int32, (K, 1), 0)
        ci = jax.lax.broadcasted_iota(jnp.int32, (1, K), 1)
        eye = (ri == ci).astype(jnp.float32)
        Sf1 = eye + S1 * r0
        Sf2 = eye + S2 * r1

        def gj_step(i, carry):
            M2, Inv, ld2c, M1, ld1c = carry
            ej_row = (ci == i).astype(jnp.float32)
            ej_col = (ri == i).astype(jnp.float32)
            rowm2 = jnp.sum(M2 * ej_col, axis=0, keepdims=True)
            rowi = jnp.sum(Inv * ej_col, axis=0, keepdims=True)
            rowm1 = jnp.sum(M1 * ej_col, axis=0, keepdims=True)
            colm2 = jnp.sum(M2 * ej_row, axis=1, keepdims=True)
            colm1 = jnp.sum(M1 * ej_row, axis=1, keepdims=True)
            p2 = jnp.sum(rowm2 * ej_row, axis=1, keepdims=True)
            p1 = jnp.sum(rowm1 * ej_row, axis=1, keepdims=True)
            rp2 = 1.0 / p2
            rp1 = 1.0 / p1
            cm2 = colm2 - ej_col
            cm1 = colm1 - ej_col
            M2 = M2 - cm2 * (rowm2 * rp2)
            Inv = Inv - cm2 * (rowi * rp2)
            M1 = M1 - cm1 * (rowm1 * rp1)
            ld2c = ld2c + (jnp.log2(p2) - 1.0)
            ld1c = ld1c + (jnp.log2(p1) - 1.0)
            return (M2, Inv, ld2c, M1, ld1c)

        ld0 = jnp.zeros((1, 1), jnp.float32)
        _, inv2, ld2, _, ld1 = jax.lax.fori_loop(
            0, K, gj_step, (Sf2, eye, ld0, Sf1, ld0))

        d = mu1 - mu2
        quad = jnp.sum(inv2 * _outer(d, d), keepdims=True)[0:1, 0:1]
        trd = jnp.sum(inv2 * (Sf1 - Sf2), keepdims=True)[0:1, 0:1]
        o_ref[...] = 0.5 * ((ld2 - ld1) + quad + trd)


def kernel(feature, label):
    labf = label.astype(jnp.float32).reshape(NSTEP, 1, BLK)
    out = pl.pallas_call(
        _fused_kernel,
        grid=(NSTEP,),
        in_specs=[
            pl.BlockSpec((BLK, K), lambda j: (j, 0)),
            pl.BlockSpec((1, 1, BLK), lambda j: (j, 0, 0)),
        ],
        out_specs=pl.BlockSpec((1, 1), lambda j: (0, 0)),
        out_shape=jax.ShapeDtypeStruct((1, 1), jnp.float32),
        scratch_shapes=[
            pltpu.VMEM((2 * K + 8, 2 * K), jnp.float32),
            pltpu.VMEM((8, K), jnp.float32),
        ],
        compiler_params=pltpu.CompilerParams(
            dimension_semantics=("arbitrary",),
        ),
    )(feature, labf)
    return out


# speedup vs baseline: 1.8580x; 1.1081x over previous
import jax
import jax.numpy as jnp
from jax.experimental import pallas as pl
from jax.experimental.pallas import tpu as pltpu

B_TOTAL = 262144
K = 128
BLK = 16384
NSTEP = B_TOTAL // BLK


def _outer(a, b):
    return jax.lax.dot_general(
        a, b, (((0,), (0,)), ((), ())), preferred_element_type=jnp.float32)


def _fused_kernel(f_ref, w_ref, o_ref, acc_ref, vec_ref):
    j = pl.program_id(0)

    @pl.when(j == 0)
    def _():
        acc_ref[...] = jnp.zeros_like(acc_ref)
        vec_ref[...] = jnp.zeros_like(vec_ref)

    HB = BLK // 16
    for half in range(16):
        f = f_ref[half * HB:(half + 1) * HB, :]
        w_row = w_ref[0, :, half * HB:(half + 1) * HB]

        h32 = pltpu.bitcast(
            pltpu.bitcast(f, jnp.int32) & jnp.int32(-65536), jnp.float32)
        l32 = f - h32
        rhs = jnp.concatenate([h32, l32], axis=1).astype(jnp.bfloat16)
        h = rhs[:, :K]

        ht = h.T
        wb = w_row.astype(jnp.bfloat16)
        hwt = ht * wb
        ones_row = jnp.ones((1, HB), jnp.bfloat16)
        zpad = jnp.zeros((6, HB), jnp.bfloat16)
        aux = jnp.concatenate([ones_row, wb, zpad], axis=0)
        lhs = jnp.concatenate([ht, hwt, aux], axis=0)

        m = jax.lax.dot_general(
            lhs, rhs, (((1,), (0,)), ((), ())),
            preferred_element_type=jnp.float32)
        acc_ref[...] += m

        n1 = jnp.sum(w_row, keepdims=True)
        vec_ref[0:1, :] += jnp.broadcast_to(n1, (1, K))

    @pl.when(j == NSTEP - 1)
    def _():
        MH = acc_ref[:K, :]
        MW = acc_ref[K:2 * K, :]
        srow = acc_ref[2 * K:2 * K + 1, :]
        wrow = acc_ref[2 * K + 1:2 * K + 2, :]
        C = MH[:, K:]
        D = MW[:, K:]
        GA = MH[:, :K] + C + C.T
        G1w = MW[:, :K] + D + D.T
        G0 = GA - G1w
        suma_ = srow[:, :K] + srow[:, K:]
        sum1w = wrow[:, :K] + wrow[:, K:]
        sum0_ = suma_ - sum1w
        n1_ = vec_ref[0:1, 0:1]
        n0_ = float(B_TOTAL) - n1_
        r0 = 1.0 / n0_
        r1 = 1.0 / n1_
        mu1 = sum0_ * r0
        mu2 = sum1w * r1

        S1 = G0 - _outer(sum0_, sum0_) * r0
        S2 = (G1w - _outer(mu1, sum1w) - _outer(sum1w, mu1)
              + n1_ * _outer(mu1, mu1))

        ri = jax.lax.broadcasted_iota(jnp.int32, (K, 1), 0)
        ci = jax.lax.broadcasted_iota(jnp.int32, (1, K), 1)
        eye = (ri == ci).astype(jnp.float32)
        Sf1 = eye + S1 * r0
        Sf2 = eye + S2 * r1

        def gj_step(i, carry):
            M2, Inv, ld2c, M1, ld1c = carry
            ej_row = (ci == i).astype(jnp.float32)
            ej_col = (ri == i).astype(jnp.float32)
            rowm2 = jnp.sum(M2 * ej_col, axis=0, keepdims=True)
            rowi = jnp.sum(Inv * ej_col, axis=0, keepdims=True)
            rowm1 = jnp.sum(M1 * ej_col, axis=0, keepdims=True)
            colm2 = jnp.sum(M2 * ej_row, axis=1, keepdims=True)
            colm1 = jnp.sum(M1 * ej_row, axis=1, keepdims=True)
            p2 = jnp.sum(rowm2 * ej_row, axis=1, keepdims=True)
            p1 = jnp.sum(rowm1 * ej_row, axis=1, keepdims=True)
            rp2 = 1.0 / p2
            rp1 = 1.0 / p1
            cm2 = colm2 - ej_col
            cm1 = colm1 - ej_col
            M2 = M2 - cm2 * (rowm2 * rp2)
            Inv = Inv - cm2 * (rowi * rp2)
            M1 = M1 - cm1 * (rowm1 * rp1)
            ld2c = ld2c + (jnp.log2(p2) - 1.0)
            ld1c = ld1c + (jnp.log2(p1) - 1.0)
            return (M2, Inv, ld2c, M1, ld1c)

        ld0 = jnp.zeros((1, 1), jnp.float32)
        _, inv2, ld2, _, ld1 = jax.lax.fori_loop(
            0, K, gj_step, (Sf2, eye, ld0, Sf1, ld0))

        d = mu1 - mu2
        quad = jnp.sum(inv2 * _outer(d, d), keepdims=True)[0:1, 0:1]
        trd = jnp.sum(inv2 * (Sf1 - Sf2), keepdims=True)[0:1, 0:1]
        o_ref[...] = 0.5 * ((ld2 - ld1) + quad + trd)


def kernel(feature, label):
    labf = label.astype(jnp.float32).reshape(NSTEP, 1, BLK)
    out = pl.pallas_call(
        _fused_kernel,
        grid=(NSTEP,),
        in_specs=[
            pl.BlockSpec((BLK, K), lambda j: (j, 0)),
            pl.BlockSpec((1, 1, BLK), lambda j: (j, 0, 0)),
        ],
        out_specs=pl.BlockSpec((1, 1), lambda j: (0, 0)),
        out_shape=jax.ShapeDtypeStruct((1, 1), jnp.float32),
        scratch_shapes=[
            pltpu.VMEM((2 * K + 8, 2 * K), jnp.float32),
            pltpu.VMEM((8, K), jnp.float32),
        ],
        compiler_params=pltpu.CompilerParams(
            dimension_semantics=("arbitrary",),
        ),
    )(feature, labf)
    return out


# speedup vs baseline: 1.9337x; 1.0407x over previous
import jax
import jax.numpy as jnp
from jax.experimental import pallas as pl
from jax.experimental.pallas import tpu as pltpu

B_TOTAL = 262144
K = 128
BLK = 32768
NSTEP = B_TOTAL // BLK


def _outer(a, b):
    return jax.lax.dot_general(
        a, b, (((0,), (0,)), ((), ())), preferred_element_type=jnp.float32)


def _fused_kernel(f_ref, w_ref, o_ref, acc_ref, vec_ref):
    j = pl.program_id(0)

    @pl.when(j == 0)
    def _():
        acc_ref[...] = jnp.zeros_like(acc_ref)
        vec_ref[...] = jnp.zeros_like(vec_ref)

    HB = BLK // 32
    for half in range(32):
        f = f_ref[half * HB:(half + 1) * HB, :]
        w_row = w_ref[0, :, half * HB:(half + 1) * HB]

        h32 = pltpu.bitcast(
            pltpu.bitcast(f, jnp.int32) & jnp.int32(-65536), jnp.float32)
        l32 = f - h32
        rhs = jnp.concatenate([h32, l32], axis=1).astype(jnp.bfloat16)
        h = rhs[:, :K]

        ht = h.T
        wb = w_row.astype(jnp.bfloat16)
        hwt = ht * wb
        ones_row = jnp.ones((1, HB), jnp.bfloat16)
        zpad = jnp.zeros((6, HB), jnp.bfloat16)
        aux = jnp.concatenate([ones_row, wb, zpad], axis=0)
        lhs = jnp.concatenate([ht, hwt, aux], axis=0)

        m = jax.lax.dot_general(
            lhs, rhs, (((1,), (0,)), ((), ())),
            preferred_element_type=jnp.float32)
        acc_ref[...] += m

        n1 = jnp.sum(w_row, keepdims=True)
        vec_ref[0:1, :] += jnp.broadcast_to(n1, (1, K))

    @pl.when(j == NSTEP - 1)
    def _():
        MH = acc_ref[:K, :]
        MW = acc_ref[K:2 * K, :]
        srow = acc_ref[2 * K:2 * K + 1, :]
        wrow = acc_ref[2 * K + 1:2 * K + 2, :]
        C = MH[:, K:]
        D = MW[:, K:]
        GA = MH[:, :K] + C + C.T
        G1w = MW[:, :K] + D + D.T
        G0 = GA - G1w
        suma_ = srow[:, :K] + srow[:, K:]
        sum1w = wrow[:, :K] + wrow[:, K:]
        sum0_ = suma_ - sum1w
        n1_ = vec_ref[0:1, 0:1]
        n0_ = float(B_TOTAL) - n1_
        r0 = 1.0 / n0_
        r1 = 1.0 / n1_
        mu1 = sum0_ * r0
        mu2 = sum1w * r1

        S1 = G0 - _outer(sum0_, sum0_) * r0
        S2 = (G1w - _outer(mu1, sum1w) - _outer(sum1w, mu1)
              + n1_ * _outer(mu1, mu1))

        ri = jax.lax.broadcasted_iota(jnp.int32, (K, 1), 0)
        ci = jax.lax.broadcasted_iota(jnp.int32, (1, K), 1)
        eye = (ri == ci).astype(jnp.float32)
        Sf1 = eye + S1 * r0
        Sf2 = eye + S2 * r1

        def gj_step(i, carry):
            M2, Inv, ld2c, M1, ld1c = carry
            ej_row = (ci == i).astype(jnp.float32)
            ej_col = (ri == i).astype(jnp.float32)
            rowm2 = jnp.sum(M2 * ej_col, axis=0, keepdims=True)
            rowi = jnp.sum(Inv * ej_col, axis=0, keepdims=True)
            rowm1 = jnp.sum(M1 * ej_col, axis=0, keepdims=True)
            colm2 = jnp.sum(M2 * ej_row, axis=1, keepdims=True)
            colm1 = jnp.sum(M1 * ej_row, axis=1, keepdims=True)
            p2 = jnp.sum(rowm2 * ej_row, axis=1, keepdims=True)
            p1 = jnp.sum(rowm1 * ej_row, axis=1, keepdims=True)
            rp2 = 1.0 / p2
            rp1 = 1.0 / p1
            cm2 = colm2 - ej_col
            cm1 = colm1 - ej_col
            M2 = M2 - cm2 * (rowm2 * rp2)
            Inv = Inv - cm2 * (rowi * rp2)
            M1 = M1 - cm1 * (rowm1 * rp1)
            ld2c = ld2c + (jnp.log2(p2) - 1.0)
            ld1c = ld1c + (jnp.log2(p1) - 1.0)
            return (M2, Inv, ld2c, M1, ld1c)

        ld0 = jnp.zeros((1, 1), jnp.float32)
        _, inv2, ld2, _, ld1 = jax.lax.fori_loop(
            0, K, gj_step, (Sf2, eye, ld0, Sf1, ld0))

        d = mu1 - mu2
        quad = jnp.sum(inv2 * _outer(d, d), keepdims=True)[0:1, 0:1]
        trd = jnp.sum(inv2 * (Sf1 - Sf2), keepdims=True)[0:1, 0:1]
        o_ref[...] = 0.5 * ((ld2 - ld1) + quad + trd)


def kernel(feature, label):
    labf = label.astype(jnp.float32).reshape(NSTEP, 1, BLK)
    out = pl.pallas_call(
        _fused_kernel,
        grid=(NSTEP,),
        in_specs=[
            pl.BlockSpec((BLK, K), lambda j: (j, 0)),
            pl.BlockSpec((1, 1, BLK), lambda j: (j, 0, 0)),
        ],
        out_specs=pl.BlockSpec((1, 1), lambda j: (0, 0)),
        out_shape=jax.ShapeDtypeStruct((1, 1), jnp.float32),
        scratch_shapes=[
            pltpu.VMEM((2 * K + 8, 2 * K), jnp.float32),
            pltpu.VMEM((8, K), jnp.float32),
        ],
        compiler_params=pltpu.CompilerParams(
            dimension_semantics=("arbitrary",),
        ),
    )(feature, labf)
    return out
